# baseline (speedup 1.0000x reference)
import jax
import jax.numpy as jnp
import numpy as np
from jax import lax
from jax.experimental import pallas as pl
from jax.experimental.pallas import tpu as pltpu
from jax.experimental.pallas import tpu_sc as plsc

N = 10000
E = 160000
F = 128
R = 16
NQ = 4
QF = F // NQ
NC = 2
NS = 16
L = 16

NP = 10240
BN = 640
BE = 2000
C = 16
ET = E // NS
G = ET // C
NROWS = NP // NS


def _phi_body(s_ref, sq_ref, vp_ref, w1_ref, b1_ref, w2_ref, b2_ref,
              out_ref, init_ref):
    x = s_ref[...]
    h = jnp.dot(x, w1_ref[...], preferred_element_type=jnp.float32) + b1_ref[...]
    h = h * (1.0 / (1.0 + jnp.exp(-h)))
    out_ref[:, 0:2 * F] = (
        jnp.dot(h, w2_ref[...], preferred_element_type=jnp.float32)
        + b2_ref[...])
    out_ref[:, 2 * F:2 * F + 3 * QF] = vp_ref[0]
    out_ref[:, 2 * F + 3 * QF:3 * F] = jnp.zeros((BN, QF), jnp.float32)
    init_ref[:, 0:QF] = sq_ref[0]
    init_ref[:, QF:F] = vp_ref[0]


def _unpack_body(in0_ref, in1_ref, in2_ref, in3_ref, s_ref, v_ref):
    blks = [in0_ref[...], in1_ref[...], in2_ref[...], in3_ref[...]]
    for q in range(NQ):
        s_ref[:, QF * q:QF * (q + 1)] = blks[q][:, 0:QF]
        for d in range(3):
            v_ref[:, F * d + QF * q:F * d + QF * (q + 1)] = (
                blks[q][:, QF * (d + 1):QF * (d + 2)])


def _wmix_body(r1_ref, r2_ref, r3_ref, fc1_ref, fc2_ref, fc3_ref,
               wr_ref, br_ref, out_ref):
    fc1, fc2, fc3 = fc1_ref[...], fc2_ref[...], fc3_ref[...]
    rm = r1_ref[...] * fc1 + r2_ref[...] * fc2 + r3_ref[...] * fc3
    out_ref[...] = (
        jnp.dot(rm, wr_ref[...], preferred_element_type=jnp.float32)
        + (fc1 + fc2 + fc3) * br_ref[...])


def _sc_body(tbl, wmix, ii, jj, uu, init, out,
             jj0, jj1, jq0, jq1, ii0, ii1, u0, u1, wm0, wm1, pr0, pr1,
             ms0, ms1, acc, sjj0, sjj1, sin0, sin1, ssc0, ssc1):
    c = lax.axis_index("c")
    t = lax.axis_index("s")
    node0 = t * NROWS
    JJ, JQ, IV, UV, WM, PR, MS = ([jj0, jj1], [jq0, jq1], [ii0, ii1],
                                  [u0, u1], [wm0, wm1], [pr0, pr1], [ms0, ms1])
    SJJ, SIN, SSC = [sjj0, sjj1], [sin0, sin1], [ssc0, ssc1]

    def compute_chunk(pr, wm_v, u_v, msg):
        def edge(e, carry2):
            xs = [[pr[e, pl.ds(QF * m + L * h, L)]
                   * wm_v[e, pl.ds(QF * m + L * h, L)]
                   for h in range(2)] for m in range(8)]
            vjs = [[pr[e, pl.ds(2 * F + QF * d + L * h, L)]
                    for h in range(2)] for d in range(3)]
            urow = u_v[e, pl.ds(0, L)]
            us = [[urow[3 * k + d] for d in range(3)] for k in range(3)]
            for h in range(2):
                msg[e, pl.ds(L * h, L)] = xs[0][h]
            for h in range(2):
                s_d = [xs[2][h] * us[0][d] + xs[3][h] * us[1][d]
                       + xs[4][h] * us[2][d] for d in range(3)]
                t_d = [xs[5][h] * us[0][d] + xs[6][h] * us[1][d]
                       + xs[7][h] * us[2][d] for d in range(3)]
                for d in range(3):
                    val = (vjs[d][h] * xs[1][h] + s_d[d]
                           + vjs[(d + 1) % 3][h] * t_d[(d + 2) % 3]
                           - vjs[(d + 2) % 3][h] * t_d[(d + 1) % 3])
                    msg[e, pl.ds(QF * (d + 1) + L * h, L)] = val
            return carry2

        lax.fori_loop(0, C, edge, 0)

    for p in range(2):
        q = c + 2 * p

        def issue_jj(g, b):
            pltpu.async_copy(jj.at[pl.ds(t * ET + g * C, C)], JJ[b], SJJ[b])

        def issue_inputs(g, b):
            base = t * ET + g * C
            pltpu.async_copy(tbl.at[JQ[b]], PR[b], SIN[b])
            pltpu.async_copy(wmix.at[pl.ds(q * E + base, C)], WM[b], SIN[b])
            pltpu.async_copy(uu.at[pl.ds(base, C)], UV[b], SIN[b])
            pltpu.async_copy(ii.at[pl.ds(base, C)], IV[b], SIN[b])

        def make_jq(b):
            JQ[b][...] = JJ[b][...] + q * NP

        def drain_jj(b):
            pltpu.make_async_copy(jj.at[pl.ds(0, C)], JJ[b], SJJ[b]).wait()

        def drain_inputs(b):
            pltpu.make_async_copy(tbl.at[JQ[b]], PR[b], SIN[b]).wait()
            pltpu.make_async_copy(wmix.at[pl.ds(0, C)], WM[b], SIN[b]).wait()
            pltpu.make_async_copy(uu.at[pl.ds(0, C)], UV[b], SIN[b]).wait()
            pltpu.make_async_copy(ii.at[pl.ds(0, C)], IV[b], SIN[b]).wait()

        def drain_sc(b):
            pltpu.make_async_copy(MS[b], acc.at[IV[b]], SSC[b]).wait()

        pltpu.sync_copy(init.at[pl.ds(q * NP + node0, NROWS)],
                        acc.at[pl.ds(node0, NROWS)])
        plsc.subcore_barrier()

        pltpu.sync_copy(jj.at[pl.ds(t * ET, C)], JJ[0])
        make_jq(0)
        issue_inputs(0, 0)
        issue_jj(1, 1)

        def step(g, b):
            nb = 1 - b

            @pl.when(g + 1 < G)
            def _prefetch():
                drain_jj(nb)
                make_jq(nb)
                issue_inputs(g + 1, nb)

            @pl.when(g + 2 < G)
            def _prefetch_jj():
                issue_jj(g + 2, b)

            drain_inputs(b)

            @pl.when(g >= 2)
            def _drain_scatter():
                drain_sc(b)

            compute_chunk(PR[b], WM[b], UV[b], MS[b])
            pltpu.async_copy(MS[b], acc.at[IV[b]], SSC[b], add=True)

        def chunk(g, carry):
            @pl.when(g % 2 == 0)
            def _even():
                step(g, 0)

            @pl.when(g % 2 == 1)
            def _odd():
                step(g, 1)

            return carry

        lax.fori_loop(0, G, chunk, 0)
        drain_sc((G - 2) % 2)
        drain_sc((G - 1) % 2)
        plsc.subcore_barrier()
        pltpu.sync_copy(acc.at[pl.ds(node0, NROWS)],
                        out.at[pl.ds(q * NP + node0, NROWS)])
        plsc.subcore_barrier()


_PERM = np.arange(8 * F).reshape(8, NQ, QF).transpose(1, 0, 2).reshape(-1)


def kernel(s, v, radial_embeddings_1, radial_embeddings_2, radial_embeddings_3,
           f_cut_1, f_cut_2, f_cut_3,
           unit_vectors_1, unit_vectors_2, unit_vectors_3,
           edge_index, W1, b1, W2, b2, Wr, br):
    idx = edge_index.astype(jnp.int32)
    perm = jnp.asarray(_PERM)
    w2p = W2[:, perm]
    b2p = b2[perm].reshape(1, 8 * F)
    wrp = Wr[:, perm]
    brp = br[perm].reshape(1, 8 * F)
    s2 = jnp.pad(s.reshape(N, F), ((0, NP - N), (0, 0)))
    squart = s2.reshape(NP, NQ, QF).transpose(1, 0, 2)
    vperm = jnp.pad(
        v.reshape(N, 3, NQ, QF).transpose(2, 0, 1, 3).reshape(NQ, N, 3 * QF),
        ((0, 0), (0, NP - N), (0, 0)))

    NB = NP // BN
    tbl, init = pl.pallas_call(
        _phi_body,
        grid=(NQ, NB),
        in_specs=[
            pl.BlockSpec((BN, F), lambda q, i: (i, 0)),
            pl.BlockSpec((1, BN, QF), lambda q, i: (q, i, 0)),
            pl.BlockSpec((1, BN, 3 * QF), lambda q, i: (q, i, 0)),
            pl.BlockSpec((F, F), lambda q, i: (0, 0)),
            pl.BlockSpec((1, F), lambda q, i: (0, 0)),
            pl.BlockSpec((F, 2 * F), lambda q, i: (0, q)),
            pl.BlockSpec((1, 2 * F), lambda q, i: (0, q)),
        ],
        out_specs=[pl.BlockSpec((BN, 3 * F), lambda q, i: (q * NB + i, 0)),
                   pl.BlockSpec((BN, F), lambda q, i: (q * NB + i, 0))],
        out_shape=[jax.ShapeDtypeStruct((NQ * NP, 3 * F), jnp.float32),
                   jax.ShapeDtypeStruct((NQ * NP, F), jnp.float32)],
    )(s2, squart, vperm, W1, b1.reshape(1, F), w2p, b2p)

    upack = jnp.concatenate(
        [unit_vectors_1, unit_vectors_2, unit_vectors_3,
         jnp.zeros((E, 7), jnp.float32)], axis=1)
    EB = E // BE
    wmix = pl.pallas_call(
        _wmix_body,
        grid=(NQ, EB),
        in_specs=[
            pl.BlockSpec((BE, R), lambda q, i: (i, 0)),
            pl.BlockSpec((BE, R), lambda q, i: (i, 0)),
            pl.BlockSpec((BE, R), lambda q, i: (i, 0)),
            pl.BlockSpec((BE, 1), lambda q, i: (i, 0)),
            pl.BlockSpec((BE, 1), lambda q, i: (i, 0)),
            pl.BlockSpec((BE, 1), lambda q, i: (i, 0)),
            pl.BlockSpec((R, 2 * F), lambda q, i: (0, q)),
            pl.BlockSpec((1, 2 * F), lambda q, i: (0, q)),
        ],
        out_specs=pl.BlockSpec((BE, 2 * F), lambda q, i: (q * EB + i, 0)),
        out_shape=jax.ShapeDtypeStruct((NQ * E, 2 * F), jnp.float32),
    )(radial_embeddings_1.reshape(E, R), radial_embeddings_2.reshape(E, R),
      radial_embeddings_3.reshape(E, R), f_cut_1, f_cut_2, f_cut_3,
      wrp, brp)

    mesh = plsc.VectorSubcoreMesh(core_axis_name="c", subcore_axis_name="s",
                                  num_cores=NC, num_subcores=NS)
    out = pl.kernel(
        _sc_body,
        out_type=jax.ShapeDtypeStruct((NQ * NP, F), jnp.float32),
        mesh=mesh,
        scratch_types=[
            pltpu.VMEM((C,), jnp.int32),
            pltpu.VMEM((C,), jnp.int32),
            pltpu.VMEM((C,), jnp.int32),
            pltpu.VMEM((C,), jnp.int32),
            pltpu.VMEM((C,), jnp.int32),
            pltpu.VMEM((C,), jnp.int32),
            pltpu.VMEM((C, 16), jnp.float32),
            pltpu.VMEM((C, 16), jnp.float32),
            pltpu.VMEM((C, 2 * F), jnp.float32),
            pltpu.VMEM((C, 2 * F), jnp.float32),
            pltpu.VMEM((C, 3 * F), jnp.float32),
            pltpu.VMEM((C, 3 * F), jnp.float32),
            pltpu.VMEM((C, F), jnp.float32),
            pltpu.VMEM((C, F), jnp.float32),
            pltpu.VMEM_SHARED((NP, F), jnp.float32),
            pltpu.SemaphoreType.DMA,
            pltpu.SemaphoreType.DMA,
            pltpu.SemaphoreType.DMA,
            pltpu.SemaphoreType.DMA,
            pltpu.SemaphoreType.DMA,
            pltpu.SemaphoreType.DMA,
        ],
    )(tbl, wmix, idx[0], idx[1], upack, init)

    s_new, v_new = pl.pallas_call(
        _unpack_body,
        grid=(NP // BN,),
        in_specs=[pl.BlockSpec((BN, F), lambda i, qq=q: (qq * NB + i, 0))
                  for q in range(NQ)],
        out_specs=[pl.BlockSpec((BN, F), lambda i: (i, 0)),
                   pl.BlockSpec((BN, 3 * F), lambda i: (i, 0))],
        out_shape=[jax.ShapeDtypeStruct((NP, F), jnp.float32),
                   jax.ShapeDtypeStruct((NP, 3 * F), jnp.float32)],
    )(out, out, out, out)
    return (s_new[:N].reshape(N, 1, F), v_new[:N].reshape(N, 3, F))


# speedup vs baseline: 1.3504x; 1.3504x over previous
import jax
import jax.numpy as jnp
import numpy as np
from jax import lax
from jax.experimental import pallas as pl
from jax.experimental.pallas import tpu as pltpu
from jax.experimental.pallas import tpu_sc as plsc

N = 10000
E = 160000
F = 128
R = 16
NQ = 4
QF = F // NQ
NC = 2
NS = 16
L = 16

NP = 10240
BN = 640
BE = 2000
C = 16
ET = E // NS
G = ET // C
NROWS = NP // NS


def _phi_body(s_ref, vp_ref, w1_ref, b1_ref, w2_ref, b2_ref,
              out_ref, init_ref):
    x = s_ref[...]
    h = jnp.dot(x, w1_ref[...], preferred_element_type=jnp.float32) + b1_ref[...]
    h = h * (1.0 / (1.0 + jnp.exp(-h)))
    for q in range(NQ):
        out_ref[q, :, 0:2 * F] = (
            jnp.dot(h, w2_ref[:, 256 * q:256 * (q + 1)],
                    preferred_element_type=jnp.float32)
            + b2_ref[:, 256 * q:256 * (q + 1)]
        )
        out_ref[q, :, 2 * F:2 * F + 3 * QF] = vp_ref[q]
        out_ref[q, :, 2 * F + 3 * QF:3 * F] = jnp.zeros((BN, QF), jnp.float32)
        init_ref[q, :, 0:QF] = x[:, QF * q:QF * (q + 1)]
        init_ref[q, :, QF:F] = vp_ref[q]


def _unpack_body(in_ref, s_ref, v_ref):
    blk = in_ref[...]
    for q in range(NQ):
        s_ref[:, QF * q:QF * (q + 1)] = blk[q, :, 0:QF]
        for d in range(3):
            v_ref[:, F * d + QF * q:F * d + QF * (q + 1)] = (
                blk[q, :, QF * (d + 1):QF * (d + 2)])


def _wmix_body(r1_ref, r2_ref, r3_ref, fcs_ref, wr_ref, br_ref, out_ref):
    rm = (r1_ref[...] * fcs_ref[:, 0:1]
          + r2_ref[...] * fcs_ref[:, 1:2]
          + r3_ref[...] * fcs_ref[:, 2:3])
    fs = fcs_ref[:, 3:4]
    for q in range(NQ):
        out_ref[q] = (
            jnp.dot(rm, wr_ref[:, 256 * q:256 * (q + 1)],
                    preferred_element_type=jnp.float32)
            + fs * br_ref[:, 256 * q:256 * (q + 1)]
        )


def _sc_body(tbl, wmix, ii, jj, uu, init, out,
             jj0, jj1, jq0, jq1, ii0, ii1, u0, u1, wm0, wm1, pr0, pr1,
             ms0, ms1, acc, sjj0, sjj1, sin0, sin1, ssc0, ssc1):
    c = lax.axis_index("c")
    t = lax.axis_index("s")
    node0 = t * NROWS
    JJ, JQ, IV, UV, WM, PR, MS = ([jj0, jj1], [jq0, jq1], [ii0, ii1],
                                  [u0, u1], [wm0, wm1], [pr0, pr1], [ms0, ms1])
    SJJ, SIN, SSC = [sjj0, sjj1], [sin0, sin1], [ssc0, ssc1]

    def compute_chunk(pr, wm_v, u_v, msg):
        def edge(e, carry2):
            xs = [[pr[e, pl.ds(QF * m + L * h, L)]
                   * wm_v[e, pl.ds(QF * m + L * h, L)]
                   for h in range(2)] for m in range(8)]
            vjs = [[pr[e, pl.ds(2 * F + QF * d + L * h, L)]
                    for h in range(2)] for d in range(3)]
            urow = u_v[e, pl.ds(0, L)]
            us = [[urow[3 * k + d] for d in range(3)] for k in range(3)]
            for h in range(2):
                msg[e, pl.ds(L * h, L)] = xs[0][h]
            for h in range(2):
                s_d = [xs[2][h] * us[0][d] + xs[3][h] * us[1][d]
                       + xs[4][h] * us[2][d] for d in range(3)]
                t_d = [xs[5][h] * us[0][d] + xs[6][h] * us[1][d]
                       + xs[7][h] * us[2][d] for d in range(3)]
                for d in range(3):
                    val = (vjs[d][h] * xs[1][h] + s_d[d]
                           + vjs[(d + 1) % 3][h] * t_d[(d + 2) % 3]
                           - vjs[(d + 2) % 3][h] * t_d[(d + 1) % 3])
                    msg[e, pl.ds(QF * (d + 1) + L * h, L)] = val
            return carry2

        lax.fori_loop(0, C, edge, 0)

    for p in range(2):
        q = c + 2 * p

        def issue_jj(g, b):
            pltpu.async_copy(jj.at[pl.ds(t * ET + g * C, C)], JJ[b], SJJ[b])

        def issue_inputs(g, b):
            base = t * ET + g * C
            pltpu.async_copy(tbl.at[JQ[b]], PR[b], SIN[b])
            pltpu.async_copy(wmix.at[pl.ds(q * E + base, C)], WM[b], SIN[b])
            pltpu.async_copy(uu.at[pl.ds(base, C)], UV[b], SIN[b])
            pltpu.async_copy(ii.at[pl.ds(base, C)], IV[b], SIN[b])

        def make_jq(b):
            JQ[b][...] = JJ[b][...] + q * NP

        def drain_jj(b):
            pltpu.make_async_copy(jj.at[pl.ds(0, C)], JJ[b], SJJ[b]).wait()

        def drain_inputs(b):
            pltpu.make_async_copy(tbl.at[JQ[b]], PR[b], SIN[b]).wait()
            pltpu.make_async_copy(wmix.at[pl.ds(0, C)], WM[b], SIN[b]).wait()
            pltpu.make_async_copy(uu.at[pl.ds(0, C)], UV[b], SIN[b]).wait()
            pltpu.make_async_copy(ii.at[pl.ds(0, C)], IV[b], SIN[b]).wait()

        def drain_sc(b):
            pltpu.make_async_copy(MS[b], acc.at[IV[b]], SSC[b]).wait()

        pltpu.sync_copy(init.at[pl.ds(q * NP + node0, NROWS)],
                        acc.at[pl.ds(node0, NROWS)])
        plsc.subcore_barrier()

        pltpu.sync_copy(jj.at[pl.ds(t * ET, C)], JJ[0])
        make_jq(0)
        issue_inputs(0, 0)
        issue_jj(1, 1)

        def step(g, b):
            nb = 1 - b

            @pl.when(g + 1 < G)
            def _prefetch():
                drain_jj(nb)
                make_jq(nb)
                issue_inputs(g + 1, nb)

            @pl.when(g + 2 < G)
            def _prefetch_jj():
                issue_jj(g + 2, b)

            drain_inputs(b)

            @pl.when(g >= 2)
            def _drain_scatter():
                drain_sc(b)

            compute_chunk(PR[b], WM[b], UV[b], MS[b])
            pltpu.async_copy(MS[b], acc.at[IV[b]], SSC[b], add=True)

        def chunk(g, carry):
            @pl.when(g % 2 == 0)
            def _even():
                step(g, 0)

            @pl.when(g % 2 == 1)
            def _odd():
                step(g, 1)

            return carry

        lax.fori_loop(0, G, chunk, 0)
        drain_sc((G - 2) % 2)
        drain_sc((G - 1) % 2)
        plsc.subcore_barrier()
        pltpu.sync_copy(acc.at[pl.ds(node0, NROWS)],
                        out.at[pl.ds(q * NP + node0, NROWS)])
        plsc.subcore_barrier()


_PERM = np.arange(8 * F).reshape(8, NQ, QF).transpose(1, 0, 2).reshape(-1)


def kernel(s, v, radial_embeddings_1, radial_embeddings_2, radial_embeddings_3,
           f_cut_1, f_cut_2, f_cut_3,
           unit_vectors_1, unit_vectors_2, unit_vectors_3,
           edge_index, W1, b1, W2, b2, Wr, br):
    idx = edge_index.astype(jnp.int32)
    perm = jnp.asarray(_PERM)
    w2p = W2[:, perm]
    b2p = b2[perm].reshape(1, 8 * F)
    wrp = Wr[:, perm]
    brp = br[perm].reshape(1, 8 * F)
    s2 = jnp.pad(s.reshape(N, F), ((0, NP - N), (0, 0)))
    vperm = jnp.pad(
        v.reshape(N, 3, NQ, QF).transpose(2, 0, 1, 3).reshape(NQ, N, 3 * QF),
        ((0, 0), (0, NP - N), (0, 0)))

    NB = NP // BN
    tbl, init = pl.pallas_call(
        _phi_body,
        grid=(NP // BN,),
        in_specs=[
            pl.BlockSpec((BN, F), lambda i: (i, 0)),
            pl.BlockSpec((NQ, BN, 3 * QF), lambda i: (0, i, 0)),
            pl.BlockSpec((F, F), lambda i: (0, 0)),
            pl.BlockSpec((1, F), lambda i: (0, 0)),
            pl.BlockSpec((F, 8 * F), lambda i: (0, 0)),
            pl.BlockSpec((1, 8 * F), lambda i: (0, 0)),
        ],
        out_specs=[pl.BlockSpec((NQ, BN, 3 * F), lambda i: (0, i, 0)),
                   pl.BlockSpec((NQ, BN, F), lambda i: (0, i, 0))],
        out_shape=[jax.ShapeDtypeStruct((NQ, NP, 3 * F), jnp.float32),
                   jax.ShapeDtypeStruct((NQ, NP, F), jnp.float32)],
    )(s2, vperm, W1, b1.reshape(1, F), w2p, b2p)
    tbl = tbl.reshape(NQ * NP, 3 * F)
    init = init.reshape(NQ * NP, F)

    upack = jnp.concatenate(
        [unit_vectors_1, unit_vectors_2, unit_vectors_3,
         jnp.zeros((E, 7), jnp.float32)], axis=1)
    fcs = jnp.concatenate(
        [f_cut_1, f_cut_2, f_cut_3, f_cut_1 + f_cut_2 + f_cut_3,
         jnp.zeros((E, 4), jnp.float32)], axis=1)
    wmix = pl.pallas_call(
        _wmix_body,
        grid=(E // BE,),
        in_specs=[
            pl.BlockSpec((BE, R), lambda i: (i, 0)),
            pl.BlockSpec((BE, R), lambda i: (i, 0)),
            pl.BlockSpec((BE, R), lambda i: (i, 0)),
            pl.BlockSpec((BE, 8), lambda i: (i, 0)),
            pl.BlockSpec((R, 8 * F), lambda i: (0, 0)),
            pl.BlockSpec((1, 8 * F), lambda i: (0, 0)),
        ],
        out_specs=pl.BlockSpec((NQ, BE, 2 * F), lambda i: (0, i, 0)),
        out_shape=jax.ShapeDtypeStruct((NQ, E, 2 * F), jnp.float32),
    )(radial_embeddings_1.reshape(E, R), radial_embeddings_2.reshape(E, R),
      radial_embeddings_3.reshape(E, R), fcs, wrp, brp).reshape(NQ * E, 2 * F)

    mesh = plsc.VectorSubcoreMesh(core_axis_name="c", subcore_axis_name="s",
                                  num_cores=NC, num_subcores=NS)
    out = pl.kernel(
        _sc_body,
        out_type=jax.ShapeDtypeStruct((NQ * NP, F), jnp.float32),
        mesh=mesh,
        scratch_types=[
            pltpu.VMEM((C,), jnp.int32),
            pltpu.VMEM((C,), jnp.int32),
            pltpu.VMEM((C,), jnp.int32),
            pltpu.VMEM((C,), jnp.int32),
            pltpu.VMEM((C,), jnp.int32),
            pltpu.VMEM((C,), jnp.int32),
            pltpu.VMEM((C, 16), jnp.float32),
            pltpu.VMEM((C, 16), jnp.float32),
            pltpu.VMEM((C, 2 * F), jnp.float32),
            pltpu.VMEM((C, 2 * F), jnp.float32),
            pltpu.VMEM((C, 3 * F), jnp.float32),
            pltpu.VMEM((C, 3 * F), jnp.float32),
            pltpu.VMEM((C, F), jnp.float32),
            pltpu.VMEM((C, F), jnp.float32),
            pltpu.VMEM_SHARED((NP, F), jnp.float32),
            pltpu.SemaphoreType.DMA,
            pltpu.SemaphoreType.DMA,
            pltpu.SemaphoreType.DMA,
            pltpu.SemaphoreType.DMA,
            pltpu.SemaphoreType.DMA,
            pltpu.SemaphoreType.DMA,
        ],
    )(tbl, wmix, idx[0], idx[1], upack, init)

    s_new, v_new = pl.pallas_call(
        _unpack_body,
        grid=(NP // BN,),
        in_specs=[pl.BlockSpec((NQ, BN, F), lambda i: (0, i, 0))],
        out_specs=[pl.BlockSpec((BN, F), lambda i: (i, 0)),
                   pl.BlockSpec((BN, 3 * F), lambda i: (i, 0))],
        out_shape=[jax.ShapeDtypeStruct((NP, F), jnp.float32),
                   jax.ShapeDtypeStruct((NP, 3 * F), jnp.float32)],
    )(out.reshape(NQ, NP, F))
    return (s_new[:N].reshape(N, 1, F), v_new[:N].reshape(N, 3, F))
